# Initial kernel scaffold; baseline (speedup 1.0000x reference)
import jax
import jax.numpy as jnp
from jax.experimental import pallas as pl


def kernel(inputs, table):
    raise NotImplementedError("write your pallas kernel here")


# speedup vs baseline: 1.7881x; 1.7881x over previous
import functools

import jax
import jax.numpy as jnp
from jax import lax
from jax.experimental import pallas as pl
from jax.experimental.pallas import tpu as pltpu
from jax.experimental.pallas import tpu_sc as plsc

N_ROWS = 4096
N_COLS = 200
B = N_ROWS * N_COLS
D = 128
NC = 2
NS = 16
NW = NC * NS
BPW = B // NW
C = 128
NCHUNK = BPW // C


def _embed_body(table_hbm, idx_hbm, out_hbm, idx_v, rows_v, sem):
    wid = lax.axis_index("s") * NC + lax.axis_index("c")
    base = wid * BPW

    def chunk(i, carry):
        off = base + i * C
        pltpu.sync_copy(idx_hbm.at[pl.ds(off, C)], idx_v)
        pltpu.async_copy(table_hbm.at[idx_v], rows_v, sem).wait()
        pltpu.sync_copy(rows_v, out_hbm.at[pl.ds(off, C)])
        return carry

    lax.fori_loop(0, NCHUNK, chunk, 0)


def kernel(inputs, table):
    idx = inputs.reshape(B).astype(jnp.int32)
    mesh = plsc.VectorSubcoreMesh(core_axis_name="c", subcore_axis_name="s")
    out = pl.kernel(
        _embed_body,
        mesh=mesh,
        out_type=jax.ShapeDtypeStruct((B, D), jnp.float32),
        scratch_types=[
            pltpu.VMEM((C,), jnp.int32),
            pltpu.VMEM((C, D), jnp.float32),
            pltpu.SemaphoreType.DMA,
        ],
    )(table, idx)
    return out.reshape(N_ROWS, N_COLS, D)


# speedup vs baseline: 1.7923x; 1.0023x over previous
import functools

import jax
import jax.numpy as jnp
from jax import lax
from jax.experimental import pallas as pl
from jax.experimental.pallas import tpu as pltpu
from jax.experimental.pallas import tpu_sc as plsc

N_ROWS = 4096
N_COLS = 200
B = N_ROWS * N_COLS
D = 128
NC = 2
NS = 16
NW = NC * NS
BPW = B // NW
C = 128
NCHUNK = BPW // C


NJ = NCHUNK // 2


def _embed_body(table_hbm, idx_hbm, out_hbm, idx_v, rows0, rows1,
                sg0, sg1, ss0, ss1):
    wid = lax.axis_index("s") * NC + lax.axis_index("c")
    base = wid * BPW

    pltpu.sync_copy(idx_hbm.at[wid], idx_v)

    def gather(i, buf, sem):
        return pltpu.async_copy(table_hbm.at[idx_v.at[i]], buf, sem)

    def store(i, buf, sem):
        return pltpu.async_copy(buf, out_hbm.at[pl.ds(base + i * C, C)], sem)

    def wait_gather(i, buf, sem):
        pltpu.make_async_copy(table_hbm.at[idx_v.at[i]], buf, sem).wait()

    def wait_store(i, buf, sem):
        pltpu.make_async_copy(
            buf, out_hbm.at[pl.ds(base + i * C, C)], sem).wait()

    gather(0, rows0, sg0)

    def body(j, carry):
        a = 2 * j
        b = a + 1
        wait_gather(a, rows0, sg0)
        store(a, rows0, ss0)

        @pl.when(j > 0)
        def _():
            wait_store(b - 2, rows1, ss1)

        gather(b, rows1, sg1)
        wait_gather(b, rows1, sg1)
        store(b, rows1, ss1)
        wait_store(a, rows0, ss0)

        @pl.when(j < NJ - 1)
        def _():
            gather(a + 2, rows0, sg0)

        return carry

    lax.fori_loop(0, NJ, body, 0)
    wait_store(NCHUNK - 1, rows1, ss1)


def kernel(inputs, table):
    idx = inputs.reshape(NW, NCHUNK, C).astype(jnp.int32)
    mesh = plsc.VectorSubcoreMesh(core_axis_name="c", subcore_axis_name="s")
    out = pl.kernel(
        _embed_body,
        mesh=mesh,
        out_type=jax.ShapeDtypeStruct((B, D), jnp.float32),
        scratch_types=[
            pltpu.VMEM((NCHUNK, C), jnp.int32),
            pltpu.VMEM((C, D), jnp.float32),
            pltpu.VMEM((C, D), jnp.float32),
            pltpu.SemaphoreType.DMA,
            pltpu.SemaphoreType.DMA,
            pltpu.SemaphoreType.DMA,
            pltpu.SemaphoreType.DMA,
        ],
    )(table, idx)
    return out.reshape(N_ROWS, N_COLS, D)
